# baseline (speedup 1.0000x reference)
import functools

import jax
import jax.numpy as jnp
from jax.experimental import pallas as pl
from jax.experimental.pallas import tpu as pltpu

K = 64


def _encode_topk_body(x_ref, wenc_ref, benc_ref, bdec_ref, hraw_ref, thr_ref,
                      hi_scr, lo_scr, *, n_cblks, bc):
    c = pl.program_id(1)
    xc = x_ref[...] - bdec_ref[...]
    acts = jax.lax.dot_general(
        xc, wenc_ref[...], (((1,), (1,)), ((), ())),
        preferred_element_type=jnp.float32)
    acts = acts + benc_ref[...]
    hr = jnp.maximum(acts, 0.0)
    hraw_ref[...] = hr
    bits = jax.lax.bitcast_convert_type(hr, jnp.int32)
    hi_scr[:, pl.ds(c * bc, bc)] = (bits >> 16).astype(jnp.int16)
    lo_scr[:, pl.ds(c * bc, bc)] = ((bits & 0xFFFF) - 32768).astype(jnp.int16)

    @pl.when(c == n_cblks - 1)
    def _search():
        def hbody(i, t):
            t_try = t | (jnp.int32(1) << (jnp.int32(14) - i))
            cnt = jnp.sum((hi_scr[...] >= t_try.astype(jnp.int16))
                          .astype(jnp.int32), axis=1, keepdims=True)
            return jnp.where(cnt >= K, t_try, t)

        t_hi = jax.lax.fori_loop(
            0, 15, hbody, jnp.zeros((hi_scr.shape[0], 1), jnp.int32),
            unroll=True)

        hi = hi_scr[...]
        t_hi16 = t_hi.astype(jnp.int16)
        w = jnp.where(
            hi > t_hi16, jnp.int16(32767),
            jnp.where(hi == t_hi16, lo_scr[...], jnp.int16(-32768)))
        hi_scr[...] = w

        def lbody(i, tw):
            t_try = tw | (jnp.int32(1) << (jnp.int32(15) - i))
            tb = (t_try - 32768).astype(jnp.int16)
            cnt = jnp.sum((hi_scr[...] >= tb).astype(jnp.int32), axis=1,
                          keepdims=True)
            return jnp.where(cnt >= K, t_try, tw)

        t_w = jax.lax.fori_loop(
            0, 16, lbody, jnp.zeros((hi_scr.shape[0], 1), jnp.int32),
            unroll=True)

        thr = (t_hi << 16) | t_w
        thr_ref[...] = jnp.broadcast_to(thr, thr_ref.shape)


def _decode_loss_body(hraw_ref, thr_ref, wdec_ref, bdec_ref, x_ref, h_ref,
                      xh_ref, loss_ref, *, n_kblks):
    k = pl.program_id(1)

    @pl.when(k == 0)
    def _():
        xh_ref[...] = jnp.broadcast_to(bdec_ref[...], xh_ref.shape)

    hr = hraw_ref[...]
    bits = jax.lax.bitcast_convert_type(hr, jnp.int32)
    hm = jnp.where(bits >= thr_ref[:, 0:1], hr, 0.0)
    h_ref[...] = hm
    xh_ref[...] += jax.lax.dot_general(
        hm.astype(jnp.bfloat16), wdec_ref[...],
        (((1,), (1,)), ((), ())),
        preferred_element_type=jnp.float32)

    @pl.when(k == n_kblks - 1)
    def _():
        d = xh_ref[...] - x_ref[...]
        loss_ref[...] = jnp.broadcast_to(
            jnp.sum(d * d).reshape(1, 1, 1), loss_ref.shape)


def _pick(n, pref):
    for b in (pref, pref // 2, pref // 4):
        if b and n % b == 0:
            return b
    return n


@jax.jit
def kernel(x, W_enc, b_enc, W_dec, b_dec):
    n, dm = x.shape
    s = W_enc.shape[0]
    br = _pick(n, 512)
    bc = _pick(s, 512)
    n_rblks, n_cblks = n // br, s // bc

    benc2 = b_enc.reshape(1, s)
    bdec2 = b_dec.reshape(1, dm)

    h_raw, thr = pl.pallas_call(
        functools.partial(_encode_topk_body, n_cblks=n_cblks, bc=bc),
        grid=(n_rblks, n_cblks),
        in_specs=[
            pl.BlockSpec((br, dm), lambda r, c: (r, 0)),
            pl.BlockSpec((bc, dm), lambda r, c: (c, 0)),
            pl.BlockSpec((1, bc), lambda r, c: (0, c)),
            pl.BlockSpec((1, dm), lambda r, c: (0, 0)),
        ],
        out_specs=[
            pl.BlockSpec((br, bc), lambda r, c: (r, c)),
            pl.BlockSpec((br, 128), lambda r, c: (r, 0)),
        ],
        out_shape=[
            jax.ShapeDtypeStruct((n, s), jnp.float32),
            jax.ShapeDtypeStruct((n, 128), jnp.int32),
        ],
        scratch_shapes=[
            pltpu.VMEM((br, s), jnp.int16),
            pltpu.VMEM((br, s), jnp.int16),
        ],
        compiler_params=pltpu.CompilerParams(
            dimension_semantics=("arbitrary", "arbitrary")),
    )(x, W_enc, benc2, bdec2)

    br2 = _pick(n, 1024)
    bk = _pick(s, 512)
    n_r2, n_kblks = n // br2, s // bk
    wdec_bf = W_dec.astype(jnp.bfloat16)

    h, x_hat, loss_parts = pl.pallas_call(
        functools.partial(_decode_loss_body, n_kblks=n_kblks),
        grid=(n_r2, n_kblks),
        in_specs=[
            pl.BlockSpec((br2, bk), lambda r, k: (r, k)),
            pl.BlockSpec((br2, 128), lambda r, k: (r, 0)),
            pl.BlockSpec((dm, bk), lambda r, k: (0, k)),
            pl.BlockSpec((1, dm), lambda r, k: (0, 0)),
            pl.BlockSpec((br2, dm), lambda r, k: (r, 0)),
        ],
        out_specs=[
            pl.BlockSpec((br2, bk), lambda r, k: (r, k)),
            pl.BlockSpec((br2, dm), lambda r, k: (r, 0)),
            pl.BlockSpec((1, 1, 128), lambda r, k: (r, 0, 0)),
        ],
        out_shape=[
            jax.ShapeDtypeStruct((n, s), jnp.float32),
            jax.ShapeDtypeStruct((n, dm), jnp.float32),
            jax.ShapeDtypeStruct((n_r2, 1, 128), jnp.float32),
        ],
        compiler_params=pltpu.CompilerParams(
            dimension_semantics=("arbitrary", "arbitrary")),
    )(h_raw, thr, wdec_bf, bdec2, x)

    loss = jnp.sum(loss_parts[:, 0, 0]) / n
    return (x_hat, h, loss)


# speedup vs baseline: 1.4340x; 1.4340x over previous
import functools

import jax
import jax.numpy as jnp
from jax.experimental import pallas as pl
from jax.experimental.pallas import tpu as pltpu

K = 64


def _encode_topk_body(x_ref, wenc_ref, benc_ref, bdec_ref, hraw_ref, thr_ref,
                      s0_scr, s1_scr, t_scr, *, n_rblks, n_cblks, bc,
                      passes_per_step):
    r = pl.program_id(0)
    c = pl.program_id(1)
    par = jax.lax.rem(r, 2)

    @pl.when(r < n_rblks)
    def _compute():
        xc = x_ref[...] - bdec_ref[...]
        acts = jax.lax.dot_general(
            xc, wenc_ref[...], (((1,), (1,)), ((), ())),
            preferred_element_type=jnp.float32)
        acts = acts + benc_ref[...]
        hr = jnp.maximum(acts, 0.0)
        hraw_ref[...] = hr
        bits = jax.lax.bitcast_convert_type(hr, jnp.int32)

        @pl.when(par == 0)
        def _():
            s0_scr[:, pl.ds(c * bc, bc)] = bits

        @pl.when(par == 1)
        def _():
            s1_scr[:, pl.ds(c * bc, bc)] = bits

    @pl.when(r >= 1)
    def _search():
        def run(s_scr):
            t = jnp.where(c == 0, 0, t_scr[:, 0:1]).astype(jnp.int32)
            sfull = s_scr[...]
            for q in range(passes_per_step):
                p = c * passes_per_step + q
                sh = jnp.maximum(jnp.int32(30) - p, 0)
                t_try = t | (jnp.int32(1) << sh)
                cnt = jnp.sum((sfull >= t_try).astype(jnp.int32), axis=1,
                              keepdims=True)
                t = jnp.where((cnt >= K) & (p <= 30), t_try, t)
            t_scr[...] = jnp.broadcast_to(t, t_scr.shape)

            @pl.when(c == n_cblks - 1)
            def _():
                thr_ref[...] = jnp.broadcast_to(t, thr_ref.shape)

        @pl.when(par == 1)
        def _():
            run(s0_scr)

        @pl.when(par == 0)
        def _():
            run(s1_scr)


def _decode_loss_body(hraw_ref, thr_ref, wdec_ref, bdec_ref, x_ref, h_ref,
                      xh_ref, loss_ref, *, n_kblks):
    k = pl.program_id(1)

    @pl.when(k == 0)
    def _():
        xh_ref[...] = jnp.broadcast_to(bdec_ref[...], xh_ref.shape)

    hr = hraw_ref[...]
    bits = jax.lax.bitcast_convert_type(hr, jnp.int32)
    hm = jnp.where(bits >= thr_ref[:, 0:1], hr, 0.0)
    h_ref[...] = hm
    xh_ref[...] += jax.lax.dot_general(
        hm.astype(jnp.bfloat16), wdec_ref[...],
        (((1,), (1,)), ((), ())),
        preferred_element_type=jnp.float32)

    @pl.when(k == n_kblks - 1)
    def _():
        d = xh_ref[...] - x_ref[...]
        loss_ref[...] = jnp.broadcast_to(
            jnp.sum(d * d).reshape(1, 1, 1), loss_ref.shape)


def _pick(n, pref):
    for b in (pref, pref // 2, pref // 4):
        if b and n % b == 0:
            return b
    return n


@jax.jit
def kernel(x, W_enc, b_enc, W_dec, b_dec):
    n, dm = x.shape
    s = W_enc.shape[0]
    br = _pick(n, 256)
    bc = _pick(s, 512)
    n_rblks, n_cblks = n // br, s // bc
    pps = -(-31 // n_cblks)

    benc2 = b_enc.reshape(1, s)
    bdec2 = b_dec.reshape(1, dm)

    h_raw_ext, thr = pl.pallas_call(
        functools.partial(_encode_topk_body, n_rblks=n_rblks,
                          n_cblks=n_cblks, bc=bc, passes_per_step=pps),
        grid=(n_rblks + 1, n_cblks),
        in_specs=[
            pl.BlockSpec((br, dm),
                         lambda r, c: (jnp.minimum(r, n_rblks - 1), 0)),
            pl.BlockSpec((bc, dm),
                         lambda r, c: (jnp.where(r < n_rblks, c, 0), 0)),
            pl.BlockSpec((1, bc),
                         lambda r, c: (0, jnp.where(r < n_rblks, c, 0))),
            pl.BlockSpec((1, dm), lambda r, c: (0, 0)),
        ],
        out_specs=[
            pl.BlockSpec((br, bc), lambda r, c: (r, c)),
            pl.BlockSpec((br, 128), lambda r, c: (jnp.maximum(r - 1, 0), 0)),
        ],
        out_shape=[
            jax.ShapeDtypeStruct((n + br, s), jnp.float32),
            jax.ShapeDtypeStruct((n, 128), jnp.int32),
        ],
        scratch_shapes=[
            pltpu.VMEM((br, s), jnp.int32),
            pltpu.VMEM((br, s), jnp.int32),
            pltpu.VMEM((br, 128), jnp.int32),
        ],
        compiler_params=pltpu.CompilerParams(
            dimension_semantics=("arbitrary", "arbitrary")),
    )(x, W_enc, benc2, bdec2)

    br2 = _pick(n, 1024)
    bk = _pick(s, 512)
    n_r2, n_kblks = n // br2, s // bk
    wdec_bf = W_dec.astype(jnp.bfloat16)

    h, x_hat, loss_parts = pl.pallas_call(
        functools.partial(_decode_loss_body, n_kblks=n_kblks),
        grid=(n_r2, n_kblks),
        in_specs=[
            pl.BlockSpec((br2, bk), lambda r, k: (r, k)),
            pl.BlockSpec((br2, 128), lambda r, k: (r, 0)),
            pl.BlockSpec((dm, bk), lambda r, k: (0, k)),
            pl.BlockSpec((1, dm), lambda r, k: (0, 0)),
            pl.BlockSpec((br2, dm), lambda r, k: (r, 0)),
        ],
        out_specs=[
            pl.BlockSpec((br2, bk), lambda r, k: (r, k)),
            pl.BlockSpec((br2, dm), lambda r, k: (r, 0)),
            pl.BlockSpec((1, 1, 128), lambda r, k: (r, 0, 0)),
        ],
        out_shape=[
            jax.ShapeDtypeStruct((n, s), jnp.float32),
            jax.ShapeDtypeStruct((n, dm), jnp.float32),
            jax.ShapeDtypeStruct((n_r2, 1, 128), jnp.float32),
        ],
        compiler_params=pltpu.CompilerParams(
            dimension_semantics=("arbitrary", "arbitrary")),
    )(h_raw_ext, thr, wdec_bf, bdec2, x)

    loss = jnp.sum(loss_parts[:, 0, 0]) / n
    return (x_hat, h, loss)


# speedup vs baseline: 1.4394x; 1.0038x over previous
import functools

import jax
import jax.numpy as jnp
from jax.experimental import pallas as pl
from jax.experimental.pallas import tpu as pltpu

K = 64


def _encode_topk_body(x_ref, wenc_ref, benc_ref, bdec_ref, hraw_ref, thr_ref,
                      s0_scr, s1_scr, t_scr, *, n_rblks, n_cblks, bc,
                      passes_per_step):
    r = pl.program_id(0)
    c = pl.program_id(1)
    par = jax.lax.rem(r, 2)

    @pl.when(r < n_rblks)
    def _compute():
        xc = x_ref[...] - bdec_ref[...]
        acts = jax.lax.dot_general(
            xc, wenc_ref[...], (((1,), (1,)), ((), ())),
            preferred_element_type=jnp.float32)
        acts = acts + benc_ref[...]
        hr = jnp.maximum(acts, 0.0)
        hraw_ref[...] = hr
        bits = jax.lax.bitcast_convert_type(hr, jnp.int32)

        @pl.when(par == 0)
        def _():
            s0_scr[:, pl.ds(c * bc, bc)] = bits

        @pl.when(par == 1)
        def _():
            s1_scr[:, pl.ds(c * bc, bc)] = bits

    @pl.when(r >= 1)
    def _search():
        def run(s_scr):
            t = jnp.where(c == 0, 0, t_scr[:, 0:1]).astype(jnp.int32)
            sfull = s_scr[...]
            for q in range(passes_per_step):
                p = c * passes_per_step + q
                sh = jnp.maximum(jnp.int32(30) - p, 0)
                t_try = t | (jnp.int32(1) << sh)
                cnt = jnp.sum((sfull >= t_try).astype(jnp.float32), axis=1,
                              keepdims=True)
                t = jnp.where((cnt >= K) & (p <= 30), t_try, t)
            t_scr[...] = jnp.broadcast_to(t, t_scr.shape)

            @pl.when(c == n_cblks - 1)
            def _():
                thr_ref[...] = jnp.broadcast_to(t, thr_ref.shape)

        @pl.when(par == 1)
        def _():
            run(s0_scr)

        @pl.when(par == 0)
        def _():
            run(s1_scr)


def _decode_loss_body(hraw_ref, thr_ref, wdec_ref, bdec_ref, x_ref, h_ref,
                      xh_ref, loss_ref, *, n_kblks):
    k = pl.program_id(1)

    @pl.when(k == 0)
    def _():
        xh_ref[...] = jnp.broadcast_to(bdec_ref[...], xh_ref.shape)

    hr = hraw_ref[...]
    bits = jax.lax.bitcast_convert_type(hr, jnp.int32)
    hm = jnp.where(bits >= thr_ref[:, 0:1], hr, 0.0)
    h_ref[...] = hm
    xh_ref[...] += jax.lax.dot_general(
        hm.astype(jnp.bfloat16), wdec_ref[...],
        (((1,), (1,)), ((), ())),
        preferred_element_type=jnp.float32)

    @pl.when(k == n_kblks - 1)
    def _():
        d = xh_ref[...] - x_ref[...]
        loss_ref[...] = jnp.broadcast_to(
            jnp.sum(d * d).reshape(1, 1, 1), loss_ref.shape)


def _pick(n, pref):
    for b in (pref, pref // 2, pref // 4):
        if b and n % b == 0:
            return b
    return n


@jax.jit
def kernel(x, W_enc, b_enc, W_dec, b_dec):
    n, dm = x.shape
    s = W_enc.shape[0]
    br = _pick(n, 256)
    bc = _pick(s, 512)
    n_rblks, n_cblks = n // br, s // bc
    pps = -(-31 // n_cblks)

    benc2 = b_enc.reshape(1, s)
    bdec2 = b_dec.reshape(1, dm)

    h_raw_ext, thr = pl.pallas_call(
        functools.partial(_encode_topk_body, n_rblks=n_rblks,
                          n_cblks=n_cblks, bc=bc, passes_per_step=pps),
        grid=(n_rblks + 1, n_cblks),
        in_specs=[
            pl.BlockSpec((br, dm),
                         lambda r, c: (jnp.minimum(r, n_rblks - 1), 0)),
            pl.BlockSpec((bc, dm),
                         lambda r, c: (jnp.where(r < n_rblks, c, 0), 0)),
            pl.BlockSpec((1, bc),
                         lambda r, c: (0, jnp.where(r < n_rblks, c, 0))),
            pl.BlockSpec((1, dm), lambda r, c: (0, 0)),
        ],
        out_specs=[
            pl.BlockSpec((br, bc), lambda r, c: (r, c)),
            pl.BlockSpec((br, 128), lambda r, c: (jnp.maximum(r - 1, 0), 0)),
        ],
        out_shape=[
            jax.ShapeDtypeStruct((n + br, s), jnp.float32),
            jax.ShapeDtypeStruct((n, 128), jnp.int32),
        ],
        scratch_shapes=[
            pltpu.VMEM((br, s), jnp.int32),
            pltpu.VMEM((br, s), jnp.int32),
            pltpu.VMEM((br, 128), jnp.int32),
        ],
        compiler_params=pltpu.CompilerParams(
            dimension_semantics=("arbitrary", "arbitrary")),
    )(x, W_enc, benc2, bdec2)

    br2 = _pick(n, 1024)
    bk = _pick(s, 512)
    n_r2, n_kblks = n // br2, s // bk
    wdec_bf = W_dec.astype(jnp.bfloat16)

    h, x_hat, loss_parts = pl.pallas_call(
        functools.partial(_decode_loss_body, n_kblks=n_kblks),
        grid=(n_r2, n_kblks),
        in_specs=[
            pl.BlockSpec((br2, bk), lambda r, k: (r, k)),
            pl.BlockSpec((br2, 128), lambda r, k: (r, 0)),
            pl.BlockSpec((dm, bk), lambda r, k: (0, k)),
            pl.BlockSpec((1, dm), lambda r, k: (0, 0)),
            pl.BlockSpec((br2, dm), lambda r, k: (r, 0)),
        ],
        out_specs=[
            pl.BlockSpec((br2, bk), lambda r, k: (r, k)),
            pl.BlockSpec((br2, dm), lambda r, k: (r, 0)),
            pl.BlockSpec((1, 1, 128), lambda r, k: (r, 0, 0)),
        ],
        out_shape=[
            jax.ShapeDtypeStruct((n, s), jnp.float32),
            jax.ShapeDtypeStruct((n, dm), jnp.float32),
            jax.ShapeDtypeStruct((n_r2, 1, 128), jnp.float32),
        ],
        compiler_params=pltpu.CompilerParams(
            dimension_semantics=("arbitrary", "arbitrary")),
    )(h_raw_ext, thr, wdec_bf, bdec2, x)

    loss = jnp.sum(loss_parts[:, 0, 0]) / n
    return (x_hat, h, loss)


# speedup vs baseline: 1.5316x; 1.0641x over previous
import functools

import jax
import jax.numpy as jnp
from jax.experimental import pallas as pl
from jax.experimental.pallas import tpu as pltpu

K = 64


def _encode_topk_body(x_ref, wenc_ref, benc_ref, bdec_ref, hraw_ref, thr_ref,
                      s0_scr, s1_scr, t_scr, *, n_rblks, n_cblks, bc,
                      passes_per_step):
    r = pl.program_id(0)
    c = pl.program_id(1)
    par = jax.lax.rem(r, 2)

    @pl.when(r < n_rblks)
    def _compute():
        xc = x_ref[...] - bdec_ref[...]
        acts = jax.lax.dot_general(
            xc, wenc_ref[...], (((1,), (1,)), ((), ())),
            preferred_element_type=jnp.float32)
        acts = acts + benc_ref[...]
        hr = jnp.maximum(acts, 0.0)
        hraw_ref[...] = hr
        bits = jax.lax.bitcast_convert_type(hr, jnp.int32)

        @pl.when(par == 0)
        def _():
            s0_scr[:, pl.ds(c * bc, bc)] = bits

        @pl.when(par == 1)
        def _():
            s1_scr[:, pl.ds(c * bc, bc)] = bits

    @pl.when(r >= 1)
    def _search():
        def run(s_scr):
            t = jnp.where(c == 0, 0, t_scr[:, 0:1]).astype(jnp.int32)
            sfull = s_scr[...]
            for q in range(passes_per_step):
                p = c * passes_per_step + q
                sh = jnp.maximum(jnp.int32(30) - p, 0)
                t_try = t | (jnp.int32(1) << sh)
                cnt = jnp.sum((sfull >= t_try).astype(jnp.float32), axis=1,
                              keepdims=True)
                t = jnp.where((cnt >= K) & (p <= 30), t_try, t)
            t_scr[...] = jnp.broadcast_to(t, t_scr.shape)

            @pl.when(c == n_cblks - 1)
            def _():
                thr_ref[...] = jnp.broadcast_to(t, thr_ref.shape)

        @pl.when(par == 1)
        def _():
            run(s0_scr)

        @pl.when(par == 0)
        def _():
            run(s1_scr)


def _decode_loss_body(hraw_ref, thr_ref, wdec_ref, bdec_ref, x_ref, h_ref,
                      xh_ref, loss_ref, *, n_kblks):
    k = pl.program_id(1)

    @pl.when(k == 0)
    def _():
        xh_ref[...] = jnp.broadcast_to(bdec_ref[...], xh_ref.shape)

    hr = hraw_ref[...]
    bits = jax.lax.bitcast_convert_type(hr, jnp.int32)
    hm = jnp.where(bits >= thr_ref[:, 0:1], hr, 0.0)
    h_ref[...] = hm
    xh_ref[...] += jax.lax.dot_general(
        hm.astype(jnp.bfloat16), wdec_ref[...],
        (((1,), (1,)), ((), ())),
        preferred_element_type=jnp.float32)

    @pl.when(k == n_kblks - 1)
    def _():
        d = xh_ref[...] - x_ref[...]
        loss_ref[...] = jnp.broadcast_to(
            jnp.sum(d * d).reshape(1, 1, 1), loss_ref.shape)


def _pick(n, pref):
    for b in (pref, pref // 2, pref // 4):
        if b and n % b == 0:
            return b
    return n


@jax.jit
def kernel(x, W_enc, b_enc, W_dec, b_dec):
    n, dm = x.shape
    s = W_enc.shape[0]
    br = _pick(n, 256)
    bc = _pick(s, 1024)
    n_rblks, n_cblks = n // br, s // bc
    pps = -(-31 // n_cblks)

    benc2 = b_enc.reshape(1, s)
    bdec2 = b_dec.reshape(1, dm)

    h_raw_ext, thr = pl.pallas_call(
        functools.partial(_encode_topk_body, n_rblks=n_rblks,
                          n_cblks=n_cblks, bc=bc, passes_per_step=pps),
        grid=(n_rblks + 1, n_cblks),
        in_specs=[
            pl.BlockSpec((br, dm),
                         lambda r, c: (jnp.minimum(r, n_rblks - 1), 0)),
            pl.BlockSpec((bc, dm),
                         lambda r, c: (jnp.where(r < n_rblks, c, 0), 0)),
            pl.BlockSpec((1, bc),
                         lambda r, c: (0, jnp.where(r < n_rblks, c, 0))),
            pl.BlockSpec((1, dm), lambda r, c: (0, 0)),
        ],
        out_specs=[
            pl.BlockSpec((br, bc), lambda r, c: (r, c)),
            pl.BlockSpec((br, 128), lambda r, c: (jnp.maximum(r - 1, 0), 0)),
        ],
        out_shape=[
            jax.ShapeDtypeStruct((n + br, s), jnp.float32),
            jax.ShapeDtypeStruct((n, 128), jnp.int32),
        ],
        scratch_shapes=[
            pltpu.VMEM((br, s), jnp.int32),
            pltpu.VMEM((br, s), jnp.int32),
            pltpu.VMEM((br, 128), jnp.int32),
        ],
        compiler_params=pltpu.CompilerParams(
            dimension_semantics=("arbitrary", "arbitrary")),
    )(x, W_enc, benc2, bdec2)

    br2 = _pick(n, 1024)
    bk = _pick(s, 512)
    n_r2, n_kblks = n // br2, s // bk
    wdec_bf = W_dec.astype(jnp.bfloat16)

    h, x_hat, loss_parts = pl.pallas_call(
        functools.partial(_decode_loss_body, n_kblks=n_kblks),
        grid=(n_r2, n_kblks),
        in_specs=[
            pl.BlockSpec((br2, bk), lambda r, k: (r, k)),
            pl.BlockSpec((br2, 128), lambda r, k: (r, 0)),
            pl.BlockSpec((dm, bk), lambda r, k: (0, k)),
            pl.BlockSpec((1, dm), lambda r, k: (0, 0)),
            pl.BlockSpec((br2, dm), lambda r, k: (r, 0)),
        ],
        out_specs=[
            pl.BlockSpec((br2, bk), lambda r, k: (r, k)),
            pl.BlockSpec((br2, dm), lambda r, k: (r, 0)),
            pl.BlockSpec((1, 1, 128), lambda r, k: (r, 0, 0)),
        ],
        out_shape=[
            jax.ShapeDtypeStruct((n, s), jnp.float32),
            jax.ShapeDtypeStruct((n, dm), jnp.float32),
            jax.ShapeDtypeStruct((n_r2, 1, 128), jnp.float32),
        ],
        compiler_params=pltpu.CompilerParams(
            dimension_semantics=("arbitrary", "arbitrary")),
    )(h_raw_ext, thr, wdec_bf, bdec2, x)

    loss = jnp.sum(loss_parts[:, 0, 0]) / n
    return (x_hat, h, loss)
